# baseline (speedup 1.0000x reference)
import functools

import jax
import jax.numpy as jnp
from jax import lax
from jax.experimental import pallas as pl
from jax.experimental.pallas import tpu as pltpu
from jax.experimental.pallas import tpu_sc as plsc

B, H, Q, DH = 16, 8, 16, 128
MAX_LEN = 4096
N_POSITIONS = 2048

NUM_WORKERS = 32
TOTAL_ROWS = B * H * MAX_LEN
ROWS_PER_WORKER = TOTAL_ROWS // NUM_WORKERS
GROUPS_PER_WORKER = ROWS_PER_WORKER // MAX_LEN
SCATTER_ROWS = GROUPS_PER_WORKER * Q
LANES = 16
CHUNKS_PER_ROW = DH // LANES


COPY_CHUNK = 128
N_CHUNKS = ROWS_PER_WORKER // COPY_CHUNK


def _copy_streams(streams, row0):
    def rd(src, c, buf, sem):
        return pltpu.make_async_copy(
            src.at[pl.ds(row0 + c * COPY_CHUNK, COPY_CHUNK)], buf, sem)

    def wr(dst, c, buf, sem):
        return pltpu.make_async_copy(
            buf, dst.at[pl.ds(row0 + c * COPY_CHUNK, COPY_CHUNK)], sem)

    for src, dst, b0, b1, sr0, sr1, sw0, sw1 in streams:
        rd(src, 0, b0, sr0).start()

    def body(p, carry):
        c0 = 2 * p

        for src, dst, b0, b1, sr0, sr1, sw0, sw1 in streams:
            @pl.when(p > 0)
            def _():
                wr(dst, c0 - 1, b1, sw1).wait()

            rd(src, c0 + 1, b1, sr1).start()

        for src, dst, b0, b1, sr0, sr1, sw0, sw1 in streams:
            rd(src, c0, b0, sr0).wait()
            wr(dst, c0, b0, sw0).start()

        for src, dst, b0, b1, sr0, sr1, sw0, sw1 in streams:
            @pl.when(p < N_CHUNKS // 2 - 1)
            def _():
                wr(dst, c0, b0, sw0).wait()
                rd(src, c0 + 2, b0, sr0).start()

        for src, dst, b0, b1, sr0, sr1, sw0, sw1 in streams:
            rd(src, c0 + 1, b1, sr1).wait()
            wr(dst, c0 + 1, b1, sw1).start()
        return carry

    lax.fori_loop(0, N_CHUNKS // 2, body, 0)
    for src, dst, b0, b1, sr0, sr1, sw0, sw1 in streams:
        wr(dst, N_CHUNKS - 2, b0, sw0).wait()
        wr(dst, N_CHUNKS - 1, b1, sw1).wait()


def _sc_body(ks, vs, pos, kc, vc, ko, vo,
             pos_v, ext, sidx_l, sidx_f, didx, krl, krf, vrl, vrf,
             kb0, kb1, vb0, vb1,
             sem_kr0, sem_kr1, sem_kw0, sem_kw1,
             sem_vr0, sem_vr1, sem_vw0, sem_vw1,
             sem_gkl, sem_gkf, sem_gvl, sem_gvf,
             sem_sk, sem_sv):
    c = lax.axis_index("c")
    s = lax.axis_index("s")
    wid = s * 2 + c
    row0 = wid * ROWS_PER_WORKER
    g0 = wid * GROUPS_PER_WORKER
    b = g0 // H

    pltpu.sync_copy(pos.at[pl.ds(b * Q, Q)], pos_v)
    pv = pos_v[...]
    iota = lax.iota(jnp.int32, Q)
    ext[pl.ds(Q, Q)] = jnp.full((Q,), -1, jnp.int32)
    ext[pl.ds(0, Q)] = pv
    nxt = ext[pl.ds(1, Q)]
    ext[pl.ds(0, Q)] = jnp.full((Q,), -1, jnp.int32)
    ext[pl.ds(Q, Q)] = pv
    prv = ext[pl.ds(Q - 1, Q)]
    xl = jnp.where(pv != nxt, iota, Q - 1)
    for k in (1, 2, 4, 8):
        ext[pl.ds(Q, Q)] = jnp.full((Q,), Q - 1, jnp.int32)
        ext[pl.ds(0, Q)] = xl
        xl = jnp.minimum(xl, ext[pl.ds(k, Q)])
    xf = jnp.where(pv != prv, iota, 0)
    for k in (1, 2, 4, 8):
        ext[pl.ds(0, Q)] = jnp.zeros((Q,), jnp.int32)
        ext[pl.ds(Q, Q)] = xf
        xf = jnp.maximum(xf, ext[pl.ds(Q - k, Q)])
    for j in range(GROUPS_PER_WORKER):
        g = g0 + j
        sidx_l[pl.ds(j * Q, Q)] = g * Q + xl
        sidx_f[pl.ds(j * Q, Q)] = g * Q + xf
        didx[pl.ds(j * Q, Q)] = g * MAX_LEN + pv

    g_kl = pltpu.make_async_copy(ks.at[sidx_l], krl, sem_gkl)
    g_kf = pltpu.make_async_copy(ks.at[sidx_f], krf, sem_gkf)
    g_vl = pltpu.make_async_copy(vs.at[sidx_l], vrl, sem_gvl)
    g_vf = pltpu.make_async_copy(vs.at[sidx_f], vrf, sem_gvf)
    g_kl.start()
    g_kf.start()
    g_vl.start()
    g_vf.start()
    g_kl.wait()
    g_kf.wait()
    g_vl.wait()
    g_vf.wait()

    lane_par = lax.iota(jnp.int32, LANES) & 1

    def _blend_body(t, carry):
        r = t // CHUNKS_PER_ROW
        ch = t % CHUNKS_PER_ROW
        off = ch * LANES
        want = jnp.where(ch < CHUNKS_PER_ROW // 2, 1, 0)
        m = lane_par == want
        krl[r, pl.ds(off, LANES)] = jnp.where(m, krf[r, pl.ds(off, LANES)],
                                              krl[r, pl.ds(off, LANES)])
        vrl[r, pl.ds(off, LANES)] = jnp.where(m, vrf[r, pl.ds(off, LANES)],
                                              vrl[r, pl.ds(off, LANES)])
        return carry

    lax.fori_loop(0, SCATTER_ROWS * CHUNKS_PER_ROW, _blend_body, 0)

    _copy_streams(
        [
            (kc, ko, kb0, kb1, sem_kr0, sem_kr1, sem_kw0, sem_kw1),
            (vc, vo, vb0, vb1, sem_vr0, sem_vr1, sem_vw0, sem_vw1),
        ],
        row0,
    )
    s_k = pltpu.make_async_copy(krl, ko.at[didx], sem_sk)
    s_v = pltpu.make_async_copy(vrl, vo.at[didx], sem_sv)
    s_k.start()
    s_v.start()
    s_k.wait()
    s_v.wait()


@jax.jit
def _sc_cache_update(ks_flat, vs_flat, pos_flat, kc_flat, vc_flat):
    mesh = plsc.VectorSubcoreMesh(core_axis_name="c", subcore_axis_name="s")
    run = pl.kernel(
        _sc_body,
        out_type=[
            jax.ShapeDtypeStruct((TOTAL_ROWS, DH), jnp.float32),
            jax.ShapeDtypeStruct((TOTAL_ROWS, DH), jnp.float32),
        ],
        mesh=mesh,
        scratch_types=[
            pltpu.VMEM((Q,), jnp.int32),
            pltpu.VMEM((2 * Q,), jnp.int32),
            pltpu.VMEM((SCATTER_ROWS,), jnp.int32),
            pltpu.VMEM((SCATTER_ROWS,), jnp.int32),
            pltpu.VMEM((SCATTER_ROWS,), jnp.int32),
            pltpu.VMEM((SCATTER_ROWS, DH), jnp.float32),
            pltpu.VMEM((SCATTER_ROWS, DH), jnp.float32),
            pltpu.VMEM((SCATTER_ROWS, DH), jnp.float32),
            pltpu.VMEM((SCATTER_ROWS, DH), jnp.float32),
            pltpu.VMEM((COPY_CHUNK, DH), jnp.float32),
            pltpu.VMEM((COPY_CHUNK, DH), jnp.float32),
            pltpu.VMEM((COPY_CHUNK, DH), jnp.float32),
            pltpu.VMEM((COPY_CHUNK, DH), jnp.float32),
        ] + [pltpu.SemaphoreType.DMA] * 14,
    )
    return run(ks_flat, vs_flat, pos_flat, kc_flat, vc_flat)


def kernel(key_states, value_states, position_ids, k_cache, v_cache, n_positions):
    ks_flat = key_states.reshape(B * H * Q, DH)
    vs_flat = value_states.reshape(B * H * Q, DH)
    pos_flat = position_ids.astype(jnp.int32).reshape(B * Q)
    kc_flat = k_cache.reshape(TOTAL_ROWS, DH)
    vc_flat = v_cache.reshape(TOTAL_ROWS, DH)
    k_out, v_out = _sc_cache_update(ks_flat, vs_flat, pos_flat, kc_flat, vc_flat)
    return (
        k_out.reshape(B, H, MAX_LEN, DH),
        v_out.reshape(B, H, MAX_LEN, DH),
    )


# speedup vs baseline: 1.0304x; 1.0304x over previous
import functools

import jax
import jax.numpy as jnp
from jax import lax
from jax.experimental import pallas as pl
from jax.experimental.pallas import tpu as pltpu
from jax.experimental.pallas import tpu_sc as plsc

B, H, Q, DH = 16, 8, 16, 128
MAX_LEN = 4096
N_POSITIONS = 2048

NUM_WORKERS = 32
TOTAL_ROWS = B * H * MAX_LEN
ROWS_PER_WORKER = TOTAL_ROWS // NUM_WORKERS
GROUPS_PER_WORKER = ROWS_PER_WORKER // MAX_LEN
SCATTER_ROWS = GROUPS_PER_WORKER * Q
LANES = 16
CHUNKS_PER_ROW = DH // LANES


COPY_CHUNK = 256
N_CHUNKS = ROWS_PER_WORKER // COPY_CHUNK


def _copy_stream(src, dst, row0, b0, b1, sr0, sr1, sw0, sw1):
    def rd(c, buf, sem):
        return pltpu.make_async_copy(
            src.at[pl.ds(row0 + c * COPY_CHUNK, COPY_CHUNK)], buf, sem)

    def wr(c, buf, sem):
        return pltpu.make_async_copy(
            buf, dst.at[pl.ds(row0 + c * COPY_CHUNK, COPY_CHUNK)], sem)

    rd(0, b0, sr0).start()

    def body(p, carry):
        c0 = 2 * p

        @pl.when(p > 0)
        def _():
            wr(c0 - 1, b1, sw1).wait()

        rd(c0 + 1, b1, sr1).start()
        rd(c0, b0, sr0).wait()
        wr(c0, b0, sw0).start()

        @pl.when(p < N_CHUNKS // 2 - 1)
        def _():
            wr(c0, b0, sw0).wait()
            rd(c0 + 2, b0, sr0).start()

        rd(c0 + 1, b1, sr1).wait()
        wr(c0 + 1, b1, sw1).start()
        return carry

    lax.fori_loop(0, N_CHUNKS // 2, body, 0)
    wr(N_CHUNKS - 2, b0, sw0).wait()
    wr(N_CHUNKS - 1, b1, sw1).wait()


def _sc_body(ks, vs, pos, kc, vc, ko, vo,
             pos_v, ext, sidx_l, sidx_f, didx, krl, krf, vrl, vrf,
             cb0, cb1,
             sem_r0, sem_r1, sem_w0, sem_w1, sem_gkl, sem_gkf, sem_gvl, sem_gvf,
             sem_sk, sem_sv):
    c = lax.axis_index("c")
    s = lax.axis_index("s")
    wid = s * 2 + c
    row0 = wid * ROWS_PER_WORKER
    g0 = wid * GROUPS_PER_WORKER
    b = g0 // H

    pltpu.sync_copy(pos.at[pl.ds(b * Q, Q)], pos_v)
    pv = pos_v[...]
    iota = lax.iota(jnp.int32, Q)
    ext[pl.ds(Q, Q)] = jnp.full((Q,), -1, jnp.int32)
    ext[pl.ds(0, Q)] = pv
    nxt = ext[pl.ds(1, Q)]
    ext[pl.ds(0, Q)] = jnp.full((Q,), -1, jnp.int32)
    ext[pl.ds(Q, Q)] = pv
    prv = ext[pl.ds(Q - 1, Q)]
    xl = jnp.where(pv != nxt, iota, Q - 1)
    for k in (1, 2, 4, 8):
        ext[pl.ds(Q, Q)] = jnp.full((Q,), Q - 1, jnp.int32)
        ext[pl.ds(0, Q)] = xl
        xl = jnp.minimum(xl, ext[pl.ds(k, Q)])
    xf = jnp.where(pv != prv, iota, 0)
    for k in (1, 2, 4, 8):
        ext[pl.ds(0, Q)] = jnp.zeros((Q,), jnp.int32)
        ext[pl.ds(Q, Q)] = xf
        xf = jnp.maximum(xf, ext[pl.ds(Q - k, Q)])
    for j in range(GROUPS_PER_WORKER):
        g = g0 + j
        sidx_l[pl.ds(j * Q, Q)] = g * Q + xl
        sidx_f[pl.ds(j * Q, Q)] = g * Q + xf
        didx[pl.ds(j * Q, Q)] = g * MAX_LEN + pv

    g_kl = pltpu.make_async_copy(ks.at[sidx_l], krl, sem_gkl)
    g_kf = pltpu.make_async_copy(ks.at[sidx_f], krf, sem_gkf)
    g_vl = pltpu.make_async_copy(vs.at[sidx_l], vrl, sem_gvl)
    g_vf = pltpu.make_async_copy(vs.at[sidx_f], vrf, sem_gvf)
    g_kl.start()
    g_kf.start()
    g_vl.start()
    g_vf.start()
    g_kl.wait()
    g_kf.wait()
    g_vl.wait()
    g_vf.wait()

    lane_par = lax.iota(jnp.int32, LANES) & 1

    def _blend_body(t, carry):
        r = t // CHUNKS_PER_ROW
        ch = t % CHUNKS_PER_ROW
        off = ch * LANES
        want = jnp.where(ch < CHUNKS_PER_ROW // 2, 1, 0)
        m = lane_par == want
        krl[r, pl.ds(off, LANES)] = jnp.where(m, krf[r, pl.ds(off, LANES)],
                                              krl[r, pl.ds(off, LANES)])
        vrl[r, pl.ds(off, LANES)] = jnp.where(m, vrf[r, pl.ds(off, LANES)],
                                              vrl[r, pl.ds(off, LANES)])
        return carry

    lax.fori_loop(0, SCATTER_ROWS * CHUNKS_PER_ROW, _blend_body, 0)

    _copy_stream(kc, ko, row0, cb0, cb1, sem_r0, sem_r1, sem_w0, sem_w1)
    _copy_stream(vc, vo, row0, cb0, cb1, sem_r0, sem_r1, sem_w0, sem_w1)
    s_k = pltpu.make_async_copy(krl, ko.at[didx], sem_sk)
    s_v = pltpu.make_async_copy(vrl, vo.at[didx], sem_sv)
    s_k.start()
    s_v.start()
    s_k.wait()
    s_v.wait()


@jax.jit
def _sc_cache_update(ks_flat, vs_flat, pos_flat, kc_flat, vc_flat):
    mesh = plsc.VectorSubcoreMesh(core_axis_name="c", subcore_axis_name="s")
    run = pl.kernel(
        _sc_body,
        out_type=[
            jax.ShapeDtypeStruct((TOTAL_ROWS, DH), jnp.float32),
            jax.ShapeDtypeStruct((TOTAL_ROWS, DH), jnp.float32),
        ],
        mesh=mesh,
        scratch_types=[
            pltpu.VMEM((Q,), jnp.int32),
            pltpu.VMEM((2 * Q,), jnp.int32),
            pltpu.VMEM((SCATTER_ROWS,), jnp.int32),
            pltpu.VMEM((SCATTER_ROWS,), jnp.int32),
            pltpu.VMEM((SCATTER_ROWS,), jnp.int32),
            pltpu.VMEM((SCATTER_ROWS, DH), jnp.float32),
            pltpu.VMEM((SCATTER_ROWS, DH), jnp.float32),
            pltpu.VMEM((SCATTER_ROWS, DH), jnp.float32),
            pltpu.VMEM((SCATTER_ROWS, DH), jnp.float32),
            pltpu.VMEM((COPY_CHUNK, DH), jnp.float32),
            pltpu.VMEM((COPY_CHUNK, DH), jnp.float32),
            pltpu.SemaphoreType.DMA,
            pltpu.SemaphoreType.DMA,
            pltpu.SemaphoreType.DMA,
            pltpu.SemaphoreType.DMA,
            pltpu.SemaphoreType.DMA,
            pltpu.SemaphoreType.DMA,
            pltpu.SemaphoreType.DMA,
            pltpu.SemaphoreType.DMA,
            pltpu.SemaphoreType.DMA,
            pltpu.SemaphoreType.DMA,
        ],
    )
    return run(ks_flat, vs_flat, pos_flat, kc_flat, vc_flat)


def kernel(key_states, value_states, position_ids, k_cache, v_cache, n_positions):
    ks_flat = key_states.reshape(B * H * Q, DH)
    vs_flat = value_states.reshape(B * H * Q, DH)
    pos_flat = position_ids.astype(jnp.int32).reshape(B * Q)
    kc_flat = k_cache.reshape(TOTAL_ROWS, DH)
    vc_flat = v_cache.reshape(TOTAL_ROWS, DH)
    k_out, v_out = _sc_cache_update(ks_flat, vs_flat, pos_flat, kc_flat, vc_flat)
    return (
        k_out.reshape(B, H, MAX_LEN, DH),
        v_out.reshape(B, H, MAX_LEN, DH),
    )
